# Initial kernel scaffold; baseline (speedup 1.0000x reference)
import jax
import jax.numpy as jnp
from jax.experimental import pallas as pl


def kernel(x, edge_index, batch, W_gat, att_src, att_dst, b_gat, W_gcn, b_gcn, W1, b1, W2, b2, W3, b3):
    raise NotImplementedError("write your pallas kernel here")


# speedup vs baseline: 1.0169x; 1.0169x over previous
import functools

import jax
import jax.numpy as jnp
from jax.experimental import pallas as pl


def _gat_in_kernel(x_ref, w_ref, m_ref, h_ref, a_ref):
    h = jnp.dot(x_ref[...], w_ref[...], preferred_element_type=jnp.float32)
    h_ref[...] = h
    a_ref[...] = jnp.dot(h, m_ref[...], preferred_element_type=jnp.float32)


def _gcn_mm_kernel(y_ref, bg_ref, w_ref, o_ref):
    y = jnp.maximum(y_ref[...] + bg_ref[...], 0.0)
    o_ref[...] = jnp.dot(y, w_ref[...], preferred_element_type=jnp.float32)


def _mlp_kernel(g_ref, w1_ref, b1_ref, w2_ref, b2_ref, w3_ref, b3_ref, o_ref):
    g = jnp.dot(g_ref[...], w1_ref[...], preferred_element_type=jnp.float32)
    g = jnp.maximum(g + b1_ref[...], 0.0)
    g = jnp.dot(g, w2_ref[...], preferred_element_type=jnp.float32) + b2_ref[...]
    o_ref[...] = jnp.dot(g, w3_ref[...], preferred_element_type=jnp.float32) + b3_ref[...]


def kernel(x, edge_index, batch, W_gat, att_src, att_dst, b_gat, W_gcn, b_gcn,
           W1, b1, W2, b2, W3, b3):
    n = x.shape[0]
    heads, ch = att_src.shape
    hc = heads * ch
    loop = jnp.arange(n, dtype=edge_index.dtype)
    src = jnp.concatenate([edge_index[0], loop])
    dst = jnp.concatenate([edge_index[1], loop])

    eye = jnp.eye(heads, dtype=x.dtype)
    m_src = (att_src[:, :, None] * eye[:, None, :]).reshape(hc, heads)
    m_dst = (att_dst[:, :, None] * eye[:, None, :]).reshape(hc, heads)
    m_att = jnp.concatenate([m_src, m_dst], axis=1)
    m_att = jnp.pad(m_att, ((0, 0), (0, 128 - 2 * heads)))

    blk = 512
    grid = (pl.cdiv(n, blk),)
    h, a = pl.pallas_call(
        _gat_in_kernel,
        grid=grid,
        in_specs=[
            pl.BlockSpec((blk, x.shape[1]), lambda i: (i, 0)),
            pl.BlockSpec(W_gat.shape, lambda i: (0, 0)),
            pl.BlockSpec(m_att.shape, lambda i: (0, 0)),
        ],
        out_specs=[
            pl.BlockSpec((blk, hc), lambda i: (i, 0)),
            pl.BlockSpec((blk, 128), lambda i: (i, 0)),
        ],
        out_shape=[
            jax.ShapeDtypeStruct((n, hc), jnp.float32),
            jax.ShapeDtypeStruct((n, 128), jnp.float32),
        ],
    )(x, W_gat, m_att)

    a_src = a[:, :heads]
    a_dst = a[:, heads:2 * heads]

    e = a_src[src] + a_dst[dst]
    e = jnp.where(e >= 0, e, 0.2 * e)
    e_max = jax.ops.segment_max(e, dst, num_segments=n)
    e_exp = jnp.exp(e - e_max[dst])
    denom = jax.ops.segment_sum(e_exp, dst, num_segments=n)
    alpha = e_exp / (denom[dst] + 1e-16)
    hs = h.reshape(n, heads, ch)[src]
    out = jax.ops.segment_sum(alpha[:, :, None] * hs, dst, num_segments=n)
    out = out.reshape(n, hc)

    xw = pl.pallas_call(
        _gcn_mm_kernel,
        grid=grid,
        in_specs=[
            pl.BlockSpec((blk, hc), lambda i: (i, 0)),
            pl.BlockSpec((1, hc), lambda i: (0, 0)),
            pl.BlockSpec(W_gcn.shape, lambda i: (0, 0)),
        ],
        out_specs=pl.BlockSpec((blk, hc), lambda i: (i, 0)),
        out_shape=jax.ShapeDtypeStruct((n, hc), jnp.float32),
    )(out, b_gat.reshape(1, hc), W_gcn)

    deg = jax.ops.segment_sum(jnp.ones(src.shape, dtype=x.dtype), dst,
                              num_segments=n)
    dis = jnp.where(deg > 0, jax.lax.rsqrt(jnp.maximum(deg, 1e-12)), 0.0)
    norm = dis[src] * dis[dst]
    x1 = jax.ops.segment_sum(norm[:, None] * xw[src], dst, num_segments=n)
    x1 = jnp.maximum(x1 + b_gcn, 0.0)

    nb = 128
    gmax = jax.ops.segment_max(x1, batch, num_segments=nb)
    gsum = jax.ops.segment_sum(x1, batch, num_segments=nb)
    cnt = jax.ops.segment_sum(jnp.ones((n,), dtype=x.dtype), batch,
                              num_segments=nb)
    gmean = gsum / jnp.maximum(cnt, 1.0)[:, None]
    g = jnp.concatenate([gmax, gmean], axis=1)

    out_mlp = pl.pallas_call(
        _mlp_kernel,
        grid=(1,),
        in_specs=[
            pl.BlockSpec(g.shape, lambda i: (0, 0)),
            pl.BlockSpec(W1.shape, lambda i: (0, 0)),
            pl.BlockSpec((1, b1.shape[0]), lambda i: (0, 0)),
            pl.BlockSpec(W2.shape, lambda i: (0, 0)),
            pl.BlockSpec((1, b2.shape[0]), lambda i: (0, 0)),
            pl.BlockSpec(W3.shape, lambda i: (0, 0)),
            pl.BlockSpec((1, b3.shape[0]), lambda i: (0, 0)),
        ],
        out_specs=pl.BlockSpec((nb, W3.shape[1]), lambda i: (0, 0)),
        out_shape=jax.ShapeDtypeStruct((nb, W3.shape[1]), jnp.float32),
    )(g, W1, b1.reshape(1, -1), W2, b2.reshape(1, -1), W3, b3.reshape(1, -1))
    return out_mlp


# speedup vs baseline: 1.1139x; 1.0955x over previous
import functools

import jax
import jax.numpy as jnp
from jax.experimental import pallas as pl


def _gat_in_kernel(x_ref, w_ref, m_ref, h_ref, a_ref):
    h = jnp.dot(x_ref[...], w_ref[...], preferred_element_type=jnp.float32)
    h_ref[...] = h
    a_ref[...] = jnp.dot(h, m_ref[...], preferred_element_type=jnp.float32)


def _gcn_mm_kernel(y_ref, bg_ref, w_ref, o_ref):
    y = jnp.maximum(y_ref[...] + bg_ref[...], 0.0)
    o_ref[...] = jnp.dot(y, w_ref[...], preferred_element_type=jnp.float32)


def _mlp_kernel(g_ref, w1_ref, b1_ref, w2_ref, b2_ref, w3_ref, b3_ref, o_ref):
    g = jnp.dot(g_ref[...], w1_ref[...], preferred_element_type=jnp.float32)
    g = jnp.maximum(g + b1_ref[...], 0.0)
    g = jnp.dot(g, w2_ref[...], preferred_element_type=jnp.float32) + b2_ref[...]
    o_ref[...] = jnp.dot(g, w3_ref[...], preferred_element_type=jnp.float32) + b3_ref[...]


def kernel(x, edge_index, batch, W_gat, att_src, att_dst, b_gat, W_gcn, b_gcn,
           W1, b1, W2, b2, W3, b3):
    n = x.shape[0]
    heads, ch = att_src.shape
    hc = heads * ch
    loop = jnp.arange(n, dtype=edge_index.dtype)
    src = jnp.concatenate([edge_index[0], loop])
    dst = jnp.concatenate([edge_index[1], loop])

    eye = jnp.eye(heads, dtype=x.dtype)
    m_src = (att_src[:, :, None] * eye[:, None, :]).reshape(hc, heads)
    m_dst = (att_dst[:, :, None] * eye[:, None, :]).reshape(hc, heads)
    m_att = jnp.concatenate([m_src, m_dst], axis=1)
    m_att = jnp.pad(m_att, ((0, 0), (0, 128 - 2 * heads)))

    blk = 512
    grid = (pl.cdiv(n, blk),)
    h, a = pl.pallas_call(
        _gat_in_kernel,
        grid=grid,
        in_specs=[
            pl.BlockSpec((blk, x.shape[1]), lambda i: (i, 0)),
            pl.BlockSpec(W_gat.shape, lambda i: (0, 0)),
            pl.BlockSpec(m_att.shape, lambda i: (0, 0)),
        ],
        out_specs=[
            pl.BlockSpec((blk, hc), lambda i: (i, 0)),
            pl.BlockSpec((blk, 128), lambda i: (i, 0)),
        ],
        out_shape=[
            jax.ShapeDtypeStruct((n, hc), jnp.float32),
            jax.ShapeDtypeStruct((n, 128), jnp.float32),
        ],
    )(x, W_gat, m_att)

    a_src = a[:, :heads]
    a_dst = a[:, heads:2 * heads]

    e = a_src[src] + a_dst[dst]
    e = jnp.where(e >= 0, e, 0.2 * e)
    e_max = jax.ops.segment_max(e, dst, num_segments=n)
    e_exp = jnp.exp(e - e_max[dst])
    denom = jax.ops.segment_sum(e_exp, dst, num_segments=n)
    alpha = e_exp / (denom[dst] + 1e-16)
    hs = h.reshape(n, heads, ch)[src]
    out = jax.ops.segment_sum(alpha[:, :, None] * hs, dst, num_segments=n)
    out = out.reshape(n, hc)

    xw = pl.pallas_call(
        _gcn_mm_kernel,
        grid=grid,
        in_specs=[
            pl.BlockSpec((blk, hc), lambda i: (i, 0)),
            pl.BlockSpec((1, hc), lambda i: (0, 0)),
            pl.BlockSpec(W_gcn.shape, lambda i: (0, 0)),
        ],
        out_specs=pl.BlockSpec((blk, hc), lambda i: (i, 0)),
        out_shape=jax.ShapeDtypeStruct((n, hc), jnp.float32),
    )(out, b_gat.reshape(1, hc), W_gcn)

    deg = jax.ops.segment_sum(jnp.ones(src.shape, dtype=x.dtype), dst,
                              num_segments=n)
    dis = jnp.where(deg > 0, jax.lax.rsqrt(jnp.maximum(deg, 1e-12)), 0.0)
    xw_s = dis[:, None] * xw
    x1 = jax.ops.segment_sum(xw_s[src], dst, num_segments=n)
    x1 = dis[:, None] * x1
    x1 = jnp.maximum(x1 + b_gcn, 0.0)

    nb = 128
    gmax = jax.ops.segment_max(x1, batch, num_segments=nb,
                               indices_are_sorted=True)
    gsum = jax.ops.segment_sum(x1, batch, num_segments=nb,
                               indices_are_sorted=True)
    cnt = jax.ops.segment_sum(jnp.ones((n,), dtype=x.dtype), batch,
                              num_segments=nb, indices_are_sorted=True)
    gmean = gsum / jnp.maximum(cnt, 1.0)[:, None]
    g = jnp.concatenate([gmax, gmean], axis=1)

    out_mlp = pl.pallas_call(
        _mlp_kernel,
        grid=(1,),
        in_specs=[
            pl.BlockSpec(g.shape, lambda i: (0, 0)),
            pl.BlockSpec(W1.shape, lambda i: (0, 0)),
            pl.BlockSpec((1, b1.shape[0]), lambda i: (0, 0)),
            pl.BlockSpec(W2.shape, lambda i: (0, 0)),
            pl.BlockSpec((1, b2.shape[0]), lambda i: (0, 0)),
            pl.BlockSpec(W3.shape, lambda i: (0, 0)),
            pl.BlockSpec((1, b3.shape[0]), lambda i: (0, 0)),
        ],
        out_specs=pl.BlockSpec((nb, W3.shape[1]), lambda i: (0, 0)),
        out_shape=jax.ShapeDtypeStruct((nb, W3.shape[1]), jnp.float32),
    )(g, W1, b1.reshape(1, -1), W2, b2.reshape(1, -1), W3, b3.reshape(1, -1))
    return out_mlp


# speedup vs baseline: 1.1285x; 1.0131x over previous
import functools

import jax
import jax.numpy as jnp
from jax.experimental import pallas as pl


def _gat_in_kernel(x_ref, w_ref, m_ref, h_ref, a_ref):
    h = jnp.dot(x_ref[...], w_ref[...], preferred_element_type=jnp.float32)
    h_ref[...] = h
    a_ref[...] = jnp.dot(h, m_ref[...], preferred_element_type=jnp.float32)


def _gcn_mm_kernel(y_ref, r_ref, bg_ref, w_ref, o_ref):
    y = jnp.maximum(y_ref[...] * r_ref[...] + bg_ref[...], 0.0)
    o_ref[...] = jnp.dot(y, w_ref[...], preferred_element_type=jnp.float32)


def _mlp_kernel(g_ref, w1_ref, b1_ref, w2_ref, b2_ref, w3_ref, b3_ref, o_ref):
    g = jnp.dot(g_ref[...], w1_ref[...], preferred_element_type=jnp.float32)
    g = jnp.maximum(g + b1_ref[...], 0.0)
    g = jnp.dot(g, w2_ref[...], preferred_element_type=jnp.float32) + b2_ref[...]
    o_ref[...] = jnp.dot(g, w3_ref[...], preferred_element_type=jnp.float32) + b3_ref[...]


def kernel(x, edge_index, batch, W_gat, att_src, att_dst, b_gat, W_gcn, b_gcn,
           W1, b1, W2, b2, W3, b3):
    n = x.shape[0]
    heads, ch = att_src.shape
    hc = heads * ch
    loop = jnp.arange(n, dtype=edge_index.dtype)
    src = jnp.concatenate([edge_index[0], loop])
    dst = jnp.concatenate([edge_index[1], loop])

    eye = jnp.eye(heads, dtype=x.dtype)
    m_src = (att_src[:, :, None] * eye[:, None, :]).reshape(hc, heads)
    m_dst = (att_dst[:, :, None] * eye[:, None, :]).reshape(hc, heads)
    m_att = jnp.concatenate([m_src, m_dst], axis=1)
    m_att = jnp.pad(m_att, ((0, 0), (0, 128 - 2 * heads)))

    blk = 512
    grid = (pl.cdiv(n, blk),)
    h, a = pl.pallas_call(
        _gat_in_kernel,
        grid=grid,
        in_specs=[
            pl.BlockSpec((blk, x.shape[1]), lambda i: (i, 0)),
            pl.BlockSpec(W_gat.shape, lambda i: (0, 0)),
            pl.BlockSpec(m_att.shape, lambda i: (0, 0)),
        ],
        out_specs=[
            pl.BlockSpec((blk, hc), lambda i: (i, 0)),
            pl.BlockSpec((blk, 128), lambda i: (i, 0)),
        ],
        out_shape=[
            jax.ShapeDtypeStruct((n, hc), jnp.float32),
            jax.ShapeDtypeStruct((n, 128), jnp.float32),
        ],
    )(x, W_gat, m_att)

    a_src = a[:, :heads]
    a_dst = a[:, heads:2 * heads]

    e = a_src[src] + a_dst[dst]
    e = jnp.where(e >= 0, e, 0.2 * e)
    e_max = jax.ops.segment_max(e, dst, num_segments=n)
    e_exp = jnp.exp(e - e_max[dst])
    denom = jax.ops.segment_sum(e_exp, dst, num_segments=n)
    hs = h.reshape(n, heads, ch)[src]
    out = jax.ops.segment_sum(e_exp[:, :, None] * hs, dst, num_segments=n)
    out = out.reshape(n, hc)
    recip = 1.0 / (denom + 1e-16)
    recip = jnp.broadcast_to(recip[:, :, None], (n, heads, ch)).reshape(n, hc)

    xw = pl.pallas_call(
        _gcn_mm_kernel,
        grid=grid,
        in_specs=[
            pl.BlockSpec((blk, hc), lambda i: (i, 0)),
            pl.BlockSpec((blk, hc), lambda i: (i, 0)),
            pl.BlockSpec((1, hc), lambda i: (0, 0)),
            pl.BlockSpec(W_gcn.shape, lambda i: (0, 0)),
        ],
        out_specs=pl.BlockSpec((blk, hc), lambda i: (i, 0)),
        out_shape=jax.ShapeDtypeStruct((n, hc), jnp.float32),
    )(out, recip, b_gat.reshape(1, hc), W_gcn)

    deg = jax.ops.segment_sum(jnp.ones(src.shape, dtype=x.dtype), dst,
                              num_segments=n)
    dis = jnp.where(deg > 0, jax.lax.rsqrt(jnp.maximum(deg, 1e-12)), 0.0)
    xw_s = dis[:, None] * xw
    x1 = jax.ops.segment_sum(xw_s[src], dst, num_segments=n)
    x1 = dis[:, None] * x1
    x1 = jnp.maximum(x1 + b_gcn, 0.0)

    nb = 128
    gmax = jax.ops.segment_max(x1, batch, num_segments=nb,
                               indices_are_sorted=True)
    gsum = jax.ops.segment_sum(x1, batch, num_segments=nb,
                               indices_are_sorted=True)
    cnt = jax.ops.segment_sum(jnp.ones((n,), dtype=x.dtype), batch,
                              num_segments=nb, indices_are_sorted=True)
    gmean = gsum / jnp.maximum(cnt, 1.0)[:, None]
    g = jnp.concatenate([gmax, gmean], axis=1)

    out_mlp = pl.pallas_call(
        _mlp_kernel,
        grid=(1,),
        in_specs=[
            pl.BlockSpec(g.shape, lambda i: (0, 0)),
            pl.BlockSpec(W1.shape, lambda i: (0, 0)),
            pl.BlockSpec((1, b1.shape[0]), lambda i: (0, 0)),
            pl.BlockSpec(W2.shape, lambda i: (0, 0)),
            pl.BlockSpec((1, b2.shape[0]), lambda i: (0, 0)),
            pl.BlockSpec(W3.shape, lambda i: (0, 0)),
            pl.BlockSpec((1, b3.shape[0]), lambda i: (0, 0)),
        ],
        out_specs=pl.BlockSpec((nb, W3.shape[1]), lambda i: (0, 0)),
        out_shape=jax.ShapeDtypeStruct((nb, W3.shape[1]), jnp.float32),
    )(g, W1, b1.reshape(1, -1), W2, b2.reshape(1, -1), W3, b3.reshape(1, -1))
    return out_mlp
